# baseline (speedup 1.0000x reference)
import functools

import jax
import jax.numpy as jnp
from jax import lax
from jax.experimental import pallas as pl
from jax.experimental.pallas import tpu as pltpu
from jax.experimental.pallas import tpu_sc as plsc

EPS = 1e-5
LANES = 16


def _tc_table(x, Wcat, bcat, n_nodes, h):
    def body(x_ref, w_ref, b_ref, t_ref):
        t_ref[...] = (
            jnp.dot(x_ref[...], w_ref[...], preferred_element_type=jnp.float32)
            + b_ref[...]
        )

    return pl.pallas_call(
        body,
        out_shape=jax.ShapeDtypeStruct((n_nodes, 2 * h), jnp.float32),
    )(x, Wcat, bcat)


def _sc_main(src, dst, T, n_nodes, n_edges, h, num_cores, num_subcores, ch):
    nw = num_cores * num_subcores
    ew = n_edges // nw
    nch = ew // ch
    npad = -(-n_nodes // (16 * LANES)) * (16 * LANES)
    seg = npad // num_subcores
    h2w = 2 * h
    mesh = plsc.VectorSubcoreMesh(core_axis_name="c", subcore_axis_name="s")

    @functools.partial(
        pl.kernel,
        mesh=mesh,
        out_type=(
            jax.ShapeDtypeStruct((n_edges // 4, h2w), jnp.float32),
            jax.ShapeDtypeStruct((n_edges // 4, h2w), jnp.float32),
            jax.ShapeDtypeStruct((n_edges,), jnp.float32),
        ),
        scratch_types=[
            pltpu.VMEM((npad,), jnp.float32),
            pltpu.VMEM((npad,), jnp.float32),
            pltpu.VMEM((ew,), jnp.int32),
            pltpu.VMEM((ew,), jnp.int32),
            pltpu.VMEM((seg,), jnp.float32),
            pltpu.VMEM((seg,), jnp.float32),
            pltpu.VMEM((ch, h2w), jnp.float32),
            pltpu.VMEM((ch, h2w), jnp.float32),
            pltpu.VMEM((ch, h2w), jnp.float32),
            pltpu.VMEM((ch, h2w), jnp.float32),
            pltpu.VMEM((ch // 2, h2w), jnp.float32),
            pltpu.VMEM((ch // 2, h2w), jnp.float32),
            pltpu.VMEM((ew,), jnp.float32),
            pltpu.VMEM_SHARED((num_subcores * npad,), jnp.float32),
            pltpu.VMEM_SHARED((num_subcores * npad,), jnp.float32),
            pltpu.VMEM_SHARED((npad,), jnp.float32),
            pltpu.VMEM_SHARED((npad,), jnp.float32),
            pltpu.SemaphoreType.DMA,
            pltpu.SemaphoreType.DMA,
            pltpu.SemaphoreType.DMA,
            pltpu.SemaphoreType.DMA,
            pltpu.SemaphoreType.DMA,
            pltpu.SemaphoreType.DMA,
        ],
        compiler_params=pltpu.CompilerParams(needs_layout_passes=False),
    )
    def main_k(src_hbm, dst_hbm, t_hbm, h1a_hbm, h1b_hbm, dp_hbm,
               ho_v, hi_v, sidx_v, didx_v, tmp_v, acc_v,
               ra0, ra1, rb0, rb1, hp0, hp1, dp_v,
               HO, HI, DGO, DGI,
               sa0, sa1, sb0, sb1, sw0, sw1):
        s = lax.axis_index("s")
        c = lax.axis_index("c")
        wid = s * num_cores + c
        half_w = nw // 2
        is_lo = wid < half_w
        rowbase = jnp.where(is_lo, wid, wid - half_w) * (ew // 2)
        zeros = jnp.zeros((LANES,), jnp.float32)
        ones = jnp.ones((LANES,), jnp.float32)
        fone = jnp.full((LANES,), 1.0, jnp.float32)

        @plsc.parallel_loop(0, npad // LANES, unroll=8)
        def _zero(i):
            ho_v[pl.ds(i * LANES, LANES)] = zeros
            hi_v[pl.ds(i * LANES, LANES)] = zeros

        for half in range(num_cores):
            b = pl.multiple_of((s * num_cores + half) * ew, ew)
            pltpu.sync_copy(src_hbm.at[pl.ds(b, ew)], sidx_v)
            pltpu.sync_copy(dst_hbm.at[pl.ds(b, ew)], didx_v)

            @pl.loop(0, ew // LANES, unroll=8)
            def _scat(j):
                sl = pl.ds(j * LANES, LANES)
                plsc.addupdate_scatter(ho_v, [sidx_v[sl]], ones)
                plsc.addupdate_scatter(hi_v, [didx_v[sl]], ones)

        srow = pl.multiple_of(s * npad, npad)
        pltpu.sync_copy(ho_v, HO.at[pl.ds(srow, npad)])
        pltpu.sync_copy(hi_v, HI.at[pl.ds(srow, npad)])
        plsc.subcore_barrier()

        off = pl.multiple_of(s * seg, seg)
        for src_sh, dst_sh in ((HO, DGO), (HI, DGI)):
            for r in range(num_subcores):
                pltpu.sync_copy(src_sh.at[pl.ds(r * npad + off, seg)], tmp_v)
                if r == 0:
                    @plsc.parallel_loop(0, seg // LANES, unroll=4)
                    def _cp(i):
                        sl = pl.ds(i * LANES, LANES)
                        acc_v[sl] = tmp_v[sl]
                else:
                    @plsc.parallel_loop(0, seg // LANES, unroll=4)
                    def _add(i):
                        sl = pl.ds(i * LANES, LANES)
                        acc_v[sl] = acc_v[sl] + tmp_v[sl]
            pltpu.sync_copy(acc_v, dst_sh.at[pl.ds(off, seg)])
        plsc.subcore_barrier()
        pltpu.sync_copy(DGO, ho_v)
        pltpu.sync_copy(DGI, hi_v)

        base0 = pl.multiple_of(wid * ew, ew)
        pltpu.sync_copy(src_hbm.at[pl.ds(base0, ew)], sidx_v)
        pltpu.sync_copy(dst_hbm.at[pl.ds(base0, ew)], didx_v)

        def fire(k, ra, rb, sa, sb):
            o = pl.multiple_of(k * ch, ch)
            pltpu.async_copy(t_hbm.at[sidx_v.at[pl.ds(o, ch)]], ra, sa)
            pltpu.async_copy(t_hbm.at[didx_v.at[pl.ds(o, ch)]], rb, sb)

        def process(k, ra, rb, sa, sb, hp, sw):
            o = pl.multiple_of(k * ch, ch)
            pltpu.make_async_copy(t_hbm.at[sidx_v.at[pl.ds(o, ch)]], ra, sa).wait()
            pltpu.make_async_copy(t_hbm.at[didx_v.at[pl.ds(o, ch)]], rb, sb).wait()
            rloc = pl.multiple_of(rowbase + k * (ch // 2), 8)
            dst_a = h1a_hbm.at[pl.ds(rloc, ch // 2)]
            dst_b = h1b_hbm.at[pl.ds(rloc, ch // 2)]

            @pl.when(jnp.logical_and(k >= 2, is_lo))
            def _():
                pltpu.make_async_copy(hp, dst_a, sw).wait()

            @pl.when(jnp.logical_and(k >= 2, jnp.logical_not(is_lo)))
            def _():
                pltpu.make_async_copy(hp, dst_b, sw).wait()

            @plsc.parallel_loop(0, ch // 2, unroll=2)
            def _row(p):
                r0 = 2 * p
                r1 = 2 * p + 1
                for j in range(h // LANES):
                    sl = pl.ds(j * LANES, LANES)
                    sh = pl.ds(h + j * LANES, LANES)
                    hp[p, sl] = ra[r0, sl] + rb[r0, sh]
                    hp[p, sh] = ra[r1, sl] + rb[r1, sh]

            for j in range(ch // LANES):
                sl = pl.ds(o + j * LANES, LANES)
                do = jnp.maximum(plsc.load_gather(ho_v, [sidx_v[sl]]), fone)
                di = jnp.maximum(plsc.load_gather(hi_v, [didx_v[sl]]), fone)
                dp_v[sl] = do * di

            @pl.when(is_lo)
            def _():
                pltpu.async_copy(hp, dst_a, sw)

            @pl.when(jnp.logical_not(is_lo))
            def _():
                pltpu.async_copy(hp, dst_b, sw)

        fire(0, ra0, rb0, sa0, sb0)

        @pl.loop(0, (nch - 1) // 2)
        def _g(g):
            k0 = 2 * g
            fire(k0 + 1, ra1, rb1, sa1, sb1)
            process(k0, ra0, rb0, sa0, sb0, hp0, sw0)
            fire(k0 + 2, ra0, rb0, sa0, sb0)
            process(k0 + 1, ra1, rb1, sa1, sb1, hp1, sw1)

        klast = nch - 1
        process(klast, ra0, rb0, sa0, sb0, hp0, sw0)
        r0d = pl.multiple_of(rowbase + klast * (ch // 2), 8)
        pltpu.make_async_copy(hp0, h1a_hbm.at[pl.ds(r0d, ch // 2)], sw0).wait()
        r1d = pl.multiple_of(rowbase + (klast - 1) * (ch // 2), 8)
        pltpu.make_async_copy(hp1, h1a_hbm.at[pl.ds(r1d, ch // 2)], sw1).wait()

        pltpu.sync_copy(dp_v, dp_hbm.at[pl.ds(base0, ew)])

    return main_k(src, dst, T)


def _tc_mlp(h1p, noise2, dp2, g1, be1, W2t, b2, g2, be2, W3r, b3,
            n_edges, h, h2, be_blk):
    nblk = (n_edges // 4) // be_blk

    def half_pipe(ht, g1v, be1v, w2v, b2v, g2v, be2v, w3v, b3v):
        m = jnp.mean(ht, axis=0, keepdims=True)
        v = jnp.mean((ht - m) ** 2, axis=0, keepdims=True)
        hn = (ht - m) * lax.rsqrt(v + EPS) * g1v + be1v
        hn = jnp.maximum(hn, 0.0)
        z = jnp.dot(w2v, hn, preferred_element_type=jnp.float32) + b2v
        m2 = jnp.mean(z, axis=0, keepdims=True)
        v2 = jnp.mean((z - m2) ** 2, axis=0, keepdims=True)
        zn = (z - m2) * lax.rsqrt(v2 + EPS) * g2v + be2v
        zn = jnp.maximum(zn, 0.0)
        return jnp.dot(w3v, zn, preferred_element_type=jnp.float32) + b3v

    def body(h1a_ref, h1b_ref, nza_ref, dpa_ref, nzb_ref, dpb_ref,
             g1_ref, be1_ref, w2_ref, b2_ref, g2_ref, be2_ref, w3_ref, b3_ref,
             outa_ref, outb_ref):
        g1v = g1_ref[...]
        be1v = be1_ref[...]
        w2v = w2_ref[...]
        b2v = b2_ref[...]
        g2v = g2_ref[...]
        be2v = be2_ref[...]
        w3v = w3_ref[...]
        b3v = b3_ref[...]
        for h1_ref, nz_ref, dp_ref, out_ref in (
            (h1a_ref, nza_ref, dpa_ref, outa_ref),
            (h1b_ref, nzb_ref, dpb_ref, outb_ref),
        ):
            ht = jnp.transpose(h1_ref[...])
            ew_even = half_pipe(ht[:h, :], g1v, be1v, w2v, b2v, g2v, be2v,
                                w3v, b3v)
            ew_odd = half_pipe(ht[h:, :], g1v, be1v, w2v, b2v, g2v, be2v,
                               w3v, b3v)
            ew = jnp.concatenate([ew_even, ew_odd], axis=0)
            nz = nz_ref[0]
            gate = jnp.log(nz) - jnp.log(1.0 - nz)
            mask = 1.0 / (1.0 + jnp.exp(-(gate + ew)))
            out_ref[0] = mask * lax.rsqrt(dp_ref[0])

    edge_spec = pl.BlockSpec((1, 2, be_blk), lambda i: (i, 0, 0))
    return pl.pallas_call(
        body,
        grid=(nblk,),
        in_specs=[
            pl.BlockSpec((be_blk, 2 * h), lambda i: (i, 0)),
            pl.BlockSpec((be_blk, 2 * h), lambda i: (i, 0)),
            edge_spec,
            edge_spec,
            edge_spec,
            edge_spec,
            pl.BlockSpec((h, 1), lambda i: (0, 0)),
            pl.BlockSpec((h, 1), lambda i: (0, 0)),
            pl.BlockSpec((h2, h), lambda i: (0, 0)),
            pl.BlockSpec((h2, 1), lambda i: (0, 0)),
            pl.BlockSpec((h2, 1), lambda i: (0, 0)),
            pl.BlockSpec((h2, 1), lambda i: (0, 0)),
            pl.BlockSpec((1, h2), lambda i: (0, 0)),
            pl.BlockSpec((1, 1), lambda i: (0, 0)),
        ],
        out_specs=(edge_spec, edge_spec),
        out_shape=(
            jax.ShapeDtypeStruct((nblk, 2, be_blk), jnp.float32),
            jax.ShapeDtypeStruct((nblk, 2, be_blk), jnp.float32),
        ),
    )(h1p[0], h1p[1], noise2[0], dp2[0], noise2[1], dp2[1],
      g1, be1, W2t, b2, g2, be2, W3r, b3)


def kernel(node_embeddings, edge_index, W1, b1, g1, be1, W2, b2, g2, be2, W3, b3):
    n_nodes, d = node_embeddings.shape
    n_edges = edge_index.shape[1]
    h = W1.shape[1]
    h2 = W2.shape[1]

    info = plsc.get_sparse_core_info()
    ch = 80

    src = edge_index[0]
    dst = edge_index[1]

    Wcat = jnp.concatenate([W1[:d], W1[d:]], axis=1)
    bcat = jnp.concatenate([b1, jnp.zeros_like(b1)]).reshape(1, 2 * h)

    T = _tc_table(node_embeddings, Wcat, bcat, n_nodes, h)
    h1a, h1b, dp = _sc_main(src, dst, T, n_nodes, n_edges, h,
                            info.num_cores, info.num_subcores, ch)

    noise = jax.random.uniform(jax.random.key(42), (n_edges,),
                               dtype=jnp.float32, minval=1e-6, maxval=1.0 - 1e-6)

    eh = n_edges // 2
    be_blk = 4000
    nblk = (eh // 2) // be_blk
    noise2 = [noise[i * eh:(i + 1) * eh].reshape(nblk, be_blk, 2)
              .transpose(0, 2, 1) for i in range(2)]
    dp2 = [dp[i * eh:(i + 1) * eh].reshape(nblk, be_blk, 2)
           .transpose(0, 2, 1) for i in range(2)]
    outa, outb = _tc_mlp(
        (h1a, h1b), noise2, dp2,
        g1.reshape(h, 1), be1.reshape(h, 1),
        W2.T, b2.reshape(h2, 1), g2.reshape(h2, 1), be2.reshape(h2, 1),
        W3.reshape(1, h2), b3.reshape(1, 1),
        n_edges, h, h2, be_blk,
    )
    return jnp.concatenate(
        [outa.transpose(0, 2, 1).reshape(eh),
         outb.transpose(0, 2, 1).reshape(eh)])


# speedup vs baseline: 1.3348x; 1.3348x over previous
import functools

import jax
import jax.numpy as jnp
from jax import lax
from jax.experimental import pallas as pl
from jax.experimental.pallas import tpu as pltpu
from jax.experimental.pallas import tpu_sc as plsc

EPS = 1e-5
LANES = 16


def _tc_table(x, Wcat, bcat, n_nodes, h):
    def body(x_ref, w_ref, b_ref, t_ref):
        t_ref[...] = (
            jnp.dot(x_ref[...], w_ref[...], preferred_element_type=jnp.float32)
            + b_ref[...]
        )

    return pl.pallas_call(
        body,
        out_shape=jax.ShapeDtypeStruct((n_nodes, 2 * h), jnp.float32),
    )(x, Wcat, bcat)


def _sc_main(src, dst, T, n_nodes, n_edges, h, num_cores, num_subcores, ch):
    nw = num_cores * num_subcores
    ew = n_edges // nw
    nch = ew // ch
    npad = -(-n_nodes // (16 * LANES)) * (16 * LANES)
    seg = npad // num_subcores
    h2w = 2 * h
    mesh = plsc.VectorSubcoreMesh(core_axis_name="c", subcore_axis_name="s")

    @functools.partial(
        pl.kernel,
        mesh=mesh,
        out_type=(
            jax.ShapeDtypeStruct((n_edges // 4 * h2w,), jnp.int32),
            jax.ShapeDtypeStruct((n_edges,), jnp.float32),
        ),
        scratch_types=[
            pltpu.VMEM((npad,), jnp.float32),
            pltpu.VMEM((npad,), jnp.float32),
            pltpu.VMEM((ew,), jnp.int32),
            pltpu.VMEM((ew,), jnp.int32),
            pltpu.VMEM((seg,), jnp.float32),
            pltpu.VMEM((seg,), jnp.float32),
            pltpu.VMEM((ch, h2w), jnp.float32),
            pltpu.VMEM((ch, h2w), jnp.float32),
            pltpu.VMEM((ch, h2w), jnp.float32),
            pltpu.VMEM((ch, h2w), jnp.float32),
            pltpu.VMEM((ch // 4 * h2w,), jnp.int32),
            pltpu.VMEM((ch // 4 * h2w,), jnp.int32),
            pltpu.VMEM((ew,), jnp.float32),
            pltpu.VMEM_SHARED((num_subcores * npad,), jnp.float32),
            pltpu.VMEM_SHARED((num_subcores * npad,), jnp.float32),
            pltpu.VMEM_SHARED((npad,), jnp.float32),
            pltpu.VMEM_SHARED((npad,), jnp.float32),
            pltpu.SemaphoreType.DMA,
            pltpu.SemaphoreType.DMA,
            pltpu.SemaphoreType.DMA,
            pltpu.SemaphoreType.DMA,
            pltpu.SemaphoreType.DMA,
            pltpu.SemaphoreType.DMA,
        ],
        compiler_params=pltpu.CompilerParams(needs_layout_passes=False),
    )
    def main_k(src_hbm, dst_hbm, t_hbm, h1_hbm, dp_hbm,
               ho_v, hi_v, sidx_v, didx_v, tmp_v, acc_v,
               ra0, ra1, rb0, rb1, hp0, hp1, dp_v,
               HO, HI, DGO, DGI,
               sa0, sa1, sb0, sb1, sw0, sw1):
        s = lax.axis_index("s")
        c = lax.axis_index("c")
        wid = s * num_cores + c
        wlen = ch // 4 * h2w
        zeros = jnp.zeros((LANES,), jnp.float32)
        ones = jnp.ones((LANES,), jnp.float32)
        fone = jnp.full((LANES,), 1.0, jnp.float32)

        @plsc.parallel_loop(0, npad // LANES, unroll=8)
        def _zero(i):
            ho_v[pl.ds(i * LANES, LANES)] = zeros
            hi_v[pl.ds(i * LANES, LANES)] = zeros

        for half in range(num_cores):
            b = pl.multiple_of((s * num_cores + half) * ew, ew)
            pltpu.sync_copy(src_hbm.at[pl.ds(b, ew)], sidx_v)
            pltpu.sync_copy(dst_hbm.at[pl.ds(b, ew)], didx_v)

            @pl.loop(0, ew // LANES, unroll=8)
            def _scat(j):
                sl = pl.ds(j * LANES, LANES)
                plsc.addupdate_scatter(ho_v, [sidx_v[sl]], ones)
                plsc.addupdate_scatter(hi_v, [didx_v[sl]], ones)

        srow = pl.multiple_of(s * npad, npad)
        pltpu.sync_copy(ho_v, HO.at[pl.ds(srow, npad)])
        pltpu.sync_copy(hi_v, HI.at[pl.ds(srow, npad)])
        plsc.subcore_barrier()

        off = pl.multiple_of(s * seg, seg)
        for src_sh, dst_sh in ((HO, DGO), (HI, DGI)):
            for r in range(num_subcores):
                pltpu.sync_copy(src_sh.at[pl.ds(r * npad + off, seg)], tmp_v)
                if r == 0:
                    @plsc.parallel_loop(0, seg // LANES, unroll=4)
                    def _cp(i):
                        sl = pl.ds(i * LANES, LANES)
                        acc_v[sl] = tmp_v[sl]
                else:
                    @plsc.parallel_loop(0, seg // LANES, unroll=4)
                    def _add(i):
                        sl = pl.ds(i * LANES, LANES)
                        acc_v[sl] = acc_v[sl] + tmp_v[sl]
            pltpu.sync_copy(acc_v, dst_sh.at[pl.ds(off, seg)])
        plsc.subcore_barrier()
        pltpu.sync_copy(DGO, ho_v)
        pltpu.sync_copy(DGI, hi_v)

        base0 = pl.multiple_of(wid * ew, ew)
        pltpu.sync_copy(src_hbm.at[pl.ds(base0, ew)], sidx_v)
        pltpu.sync_copy(dst_hbm.at[pl.ds(base0, ew)], didx_v)

        def fire(k, ra, rb, sa, sb):
            o = pl.multiple_of(k * ch, ch)
            pltpu.async_copy(t_hbm.at[sidx_v.at[pl.ds(o, ch)]], ra, sa)
            pltpu.async_copy(t_hbm.at[didx_v.at[pl.ds(o, ch)]], rb, sb)

        def process(k, ra, rb, sa, sb, hp, sw):
            o = pl.multiple_of(k * ch, ch)
            pltpu.make_async_copy(t_hbm.at[sidx_v.at[pl.ds(o, ch)]], ra, sa).wait()
            pltpu.make_async_copy(t_hbm.at[didx_v.at[pl.ds(o, ch)]], rb, sb).wait()
            woff = pl.multiple_of((base0 + k * ch) // 4 * h2w, wlen)
            h1_dst = h1_hbm.at[pl.ds(woff, wlen)]

            @pl.when(k >= 2)
            def _():
                pltpu.make_async_copy(hp, h1_dst, sw).wait()

            @plsc.parallel_loop(0, ch // 4, unroll=2)
            def _row(p):
                r0 = 4 * p
                base = p * h2w
                for j in range(h // LANES):
                    sl = pl.ds(j * LANES, LANES)
                    sh = pl.ds(h + j * LANES, LANES)
                    v0 = ra[r0, sl] + rb[r0, sh]
                    v1 = ra[r0 + 1, sl] + rb[r0 + 1, sh]
                    v2 = ra[r0 + 2, sl] + rb[r0 + 2, sh]
                    v3 = ra[r0 + 3, sl] + rb[r0 + 3, sh]
                    hp[pl.ds(base + j * LANES, LANES)] = plsc.bitcast(
                        plsc.pack(v0, v1, format=plsc.PackFormat.INTERLEAVED),
                        jnp.int32)
                    hp[pl.ds(base + h + j * LANES, LANES)] = plsc.bitcast(
                        plsc.pack(v2, v3, format=plsc.PackFormat.INTERLEAVED),
                        jnp.int32)

            for j in range(ch // LANES):
                sl = pl.ds(o + j * LANES, LANES)
                do = jnp.maximum(plsc.load_gather(ho_v, [sidx_v[sl]]), fone)
                di = jnp.maximum(plsc.load_gather(hi_v, [didx_v[sl]]), fone)
                dp_v[sl] = do * di

            pltpu.async_copy(hp, h1_dst, sw)

        fire(0, ra0, rb0, sa0, sb0)

        @pl.loop(0, (nch - 1) // 2)
        def _g(g):
            k0 = 2 * g
            fire(k0 + 1, ra1, rb1, sa1, sb1)
            process(k0, ra0, rb0, sa0, sb0, hp0, sw0)
            fire(k0 + 2, ra0, rb0, sa0, sb0)
            process(k0 + 1, ra1, rb1, sa1, sb1, hp1, sw1)

        klast = nch - 1
        process(klast, ra0, rb0, sa0, sb0, hp0, sw0)
        w0d = pl.multiple_of((base0 + klast * ch) // 4 * h2w, wlen)
        pltpu.make_async_copy(hp0, h1_hbm.at[pl.ds(w0d, wlen)], sw0).wait()
        w1d = pl.multiple_of((base0 + (klast - 1) * ch) // 4 * h2w, wlen)
        pltpu.make_async_copy(hp1, h1_hbm.at[pl.ds(w1d, wlen)], sw1).wait()

        pltpu.sync_copy(dp_v, dp_hbm.at[pl.ds(base0, ew)])

    return main_k(src, dst, T)


def _tc_mlp(h1q, noise2, dp2, g1, be1, W2t, b2, g2, be2, W3r, b3,
            n_edges, h, h2, be_blk):
    nblk = (n_edges // 4) // be_blk

    def half_pipe(ht, g1v, be1v, w2v, b2v, g2v, be2v, w3v, b3v):
        m = jnp.mean(ht, axis=0, keepdims=True)
        v = jnp.mean((ht - m) ** 2, axis=0, keepdims=True)
        hn = (ht - m) * lax.rsqrt(v + EPS) * g1v + be1v
        hn = jnp.maximum(hn, 0.0)
        z = jnp.dot(w2v, hn, preferred_element_type=jnp.float32) + b2v
        m2 = jnp.mean(z, axis=0, keepdims=True)
        v2 = jnp.mean((z - m2) ** 2, axis=0, keepdims=True)
        zn = (z - m2) * lax.rsqrt(v2 + EPS) * g2v + be2v
        zn = jnp.maximum(zn, 0.0)
        return jnp.dot(w3v, zn, preferred_element_type=jnp.float32) + b3v

    def body(h1_ref, nz_ref, dp_ref, g1_ref, be1_ref, w2_ref, b2_ref, g2_ref,
             be2_ref, w3_ref, b3_ref, out_ref):
        g1v = g1_ref[...]
        be1v = be1_ref[...]
        w2v = w2_ref[...]
        b2v = b2_ref[...]
        g2v = g2_ref[...]
        be2v = be2_ref[...]
        w3v = w3_ref[...]
        b3v = b3_ref[...]
        htq = jnp.transpose(h1_ref[...])
        lo = lax.bitcast_convert_type(
            lax.shift_left(htq, 16), jnp.float32)
        hi = lax.bitcast_convert_type(
            jnp.bitwise_and(htq, jnp.int32(-65536)), jnp.float32)
        args = (g1v, be1v, w2v, b2v, g2v, be2v, w3v, b3v)
        ew0 = half_pipe(lo[:h, :], *args)
        ew1 = half_pipe(hi[:h, :], *args)
        ew2 = half_pipe(lo[h:, :], *args)
        ew3 = half_pipe(hi[h:, :], *args)
        ew = jnp.concatenate([ew0, ew1, ew2, ew3], axis=0)
        nz = nz_ref[0]
        gate = jnp.log(nz) - jnp.log(1.0 - nz)
        mask = 1.0 / (1.0 + jnp.exp(-(gate + ew)))
        out_ref[0] = mask * lax.rsqrt(dp_ref[0])

    edge_spec = pl.BlockSpec((1, 4, be_blk), lambda i: (i, 0, 0))
    return pl.pallas_call(
        body,
        grid=(nblk,),
        in_specs=[
            pl.BlockSpec((be_blk, 2 * h), lambda i: (i, 0)),
            edge_spec,
            edge_spec,
            pl.BlockSpec((h, 1), lambda i: (0, 0)),
            pl.BlockSpec((h, 1), lambda i: (0, 0)),
            pl.BlockSpec((h2, h), lambda i: (0, 0)),
            pl.BlockSpec((h2, 1), lambda i: (0, 0)),
            pl.BlockSpec((h2, 1), lambda i: (0, 0)),
            pl.BlockSpec((h2, 1), lambda i: (0, 0)),
            pl.BlockSpec((1, h2), lambda i: (0, 0)),
            pl.BlockSpec((1, 1), lambda i: (0, 0)),
        ],
        out_specs=edge_spec,
        out_shape=jax.ShapeDtypeStruct((nblk, 4, be_blk), jnp.float32),
    )(h1q, noise2, dp2, g1, be1, W2t, b2, g2, be2, W3r, b3)


def kernel(node_embeddings, edge_index, W1, b1, g1, be1, W2, b2, g2, be2, W3, b3):
    n_nodes, d = node_embeddings.shape
    n_edges = edge_index.shape[1]
    h = W1.shape[1]
    h2 = W2.shape[1]

    info = plsc.get_sparse_core_info()
    ch = 80

    src = edge_index[0]
    dst = edge_index[1]

    Wcat = jnp.concatenate([W1[:d], W1[d:]], axis=1)
    bcat = jnp.concatenate([b1, jnp.zeros_like(b1)]).reshape(1, 2 * h)

    T = _tc_table(node_embeddings, Wcat, bcat, n_nodes, h)
    h1f, dp = _sc_main(src, dst, T, n_nodes, n_edges, h,
                       info.num_cores, info.num_subcores, ch)
    h1q = h1f.reshape(n_edges // 4, 2 * h)

    noise = jax.random.uniform(jax.random.key(42), (n_edges,),
                               dtype=jnp.float32, minval=1e-6, maxval=1.0 - 1e-6)

    be_blk = 4000
    nblk = (n_edges // 4) // be_blk
    noise2 = noise.reshape(nblk, be_blk, 4).transpose(0, 2, 1)
    dp2 = dp.reshape(nblk, be_blk, 4).transpose(0, 2, 1)
    out2 = _tc_mlp(
        h1q, noise2, dp2,
        g1.reshape(h, 1), be1.reshape(h, 1),
        W2.T, b2.reshape(h2, 1), g2.reshape(h2, 1), be2.reshape(h2, 1),
        W3.reshape(1, h2), b3.reshape(1, 1),
        n_edges, h, h2, be_blk,
    )
    return out2.transpose(0, 2, 1).reshape(n_edges)


# speedup vs baseline: 1.3488x; 1.0105x over previous
import functools

import jax
import jax.numpy as jnp
from jax import lax
from jax.experimental import pallas as pl
from jax.experimental.pallas import tpu as pltpu
from jax.experimental.pallas import tpu_sc as plsc

EPS = 1e-5
LANES = 16


def _tc_table(x, Wcat, bcat, n_nodes, h):
    def body(x_ref, w_ref, b_ref, t_ref):
        t_ref[...] = (
            jnp.dot(x_ref[...], w_ref[...], preferred_element_type=jnp.float32)
            + b_ref[...]
        )

    return pl.pallas_call(
        body,
        out_shape=jax.ShapeDtypeStruct((n_nodes, 2 * h), jnp.float32),
    )(x, Wcat, bcat)


def _sc_main(src, dst, T, n_nodes, n_edges, h, num_cores, num_subcores, ch):
    nw = num_cores * num_subcores
    ew = n_edges // nw
    nch = ew // ch
    npad = -(-n_nodes // (16 * LANES)) * (16 * LANES)
    seg = npad // num_subcores
    h2w = 2 * h
    mesh = plsc.VectorSubcoreMesh(core_axis_name="c", subcore_axis_name="s")

    @functools.partial(
        pl.kernel,
        mesh=mesh,
        out_type=(
            jax.ShapeDtypeStruct((n_edges // 4 * h2w,), jnp.int32),
            jax.ShapeDtypeStruct((n_edges,), jnp.float32),
        ),
        scratch_types=[
            pltpu.VMEM((npad,), jnp.float32),
            pltpu.VMEM((npad,), jnp.float32),
            pltpu.VMEM((ew,), jnp.int32),
            pltpu.VMEM((ew,), jnp.int32),
            pltpu.VMEM((seg,), jnp.float32),
            pltpu.VMEM((seg,), jnp.float32),
            pltpu.VMEM((ch, h2w), jnp.float32),
            pltpu.VMEM((ch, h2w), jnp.float32),
            pltpu.VMEM((ch, h2w), jnp.float32),
            pltpu.VMEM((ch, h2w), jnp.float32),
            pltpu.VMEM((ch // 4 * h2w,), jnp.int32),
            pltpu.VMEM((ch // 4 * h2w,), jnp.int32),
            pltpu.VMEM((ew,), jnp.float32),
            pltpu.VMEM_SHARED((num_subcores * npad,), jnp.float32),
            pltpu.VMEM_SHARED((num_subcores * npad,), jnp.float32),
            pltpu.VMEM_SHARED((npad,), jnp.float32),
            pltpu.VMEM_SHARED((npad,), jnp.float32),
            pltpu.SemaphoreType.DMA,
            pltpu.SemaphoreType.DMA,
            pltpu.SemaphoreType.DMA,
            pltpu.SemaphoreType.DMA,
            pltpu.SemaphoreType.DMA,
            pltpu.SemaphoreType.DMA,
        ],
        compiler_params=pltpu.CompilerParams(needs_layout_passes=False),
    )
    def main_k(src_hbm, dst_hbm, t_hbm, h1_hbm, dp_hbm,
               ho_v, hi_v, sidx_v, didx_v, tmp_v, acc_v,
               ra0, ra1, rb0, rb1, hp0, hp1, dp_v,
               HO, HI, DGO, DGI,
               sa0, sa1, sb0, sb1, sw0, sw1):
        s = lax.axis_index("s")
        c = lax.axis_index("c")
        wid = s * num_cores + c
        wlen = ch // 4 * h2w
        zeros = jnp.zeros((LANES,), jnp.float32)
        ones = jnp.ones((LANES,), jnp.float32)
        fone = jnp.full((LANES,), 1.0, jnp.float32)

        @plsc.parallel_loop(0, npad // LANES, unroll=8)
        def _zero(i):
            ho_v[pl.ds(i * LANES, LANES)] = zeros
            hi_v[pl.ds(i * LANES, LANES)] = zeros

        for half in range(num_cores):
            b = pl.multiple_of((s * num_cores + half) * ew, ew)
            pltpu.sync_copy(src_hbm.at[pl.ds(b, ew)], sidx_v)
            pltpu.sync_copy(dst_hbm.at[pl.ds(b, ew)], didx_v)

            @pl.loop(0, ew // LANES, unroll=8)
            def _scat(j):
                sl = pl.ds(j * LANES, LANES)
                plsc.addupdate_scatter(ho_v, [sidx_v[sl]], ones)
                plsc.addupdate_scatter(hi_v, [didx_v[sl]], ones)

        srow = pl.multiple_of(s * npad, npad)
        pltpu.sync_copy(ho_v, HO.at[pl.ds(srow, npad)])
        pltpu.sync_copy(hi_v, HI.at[pl.ds(srow, npad)])
        plsc.subcore_barrier()

        off = pl.multiple_of(s * seg, seg)
        for src_sh, dst_sh in ((HO, DGO), (HI, DGI)):
            for r in range(num_subcores):
                pltpu.sync_copy(src_sh.at[pl.ds(r * npad + off, seg)], tmp_v)
                if r == 0:
                    @plsc.parallel_loop(0, seg // LANES, unroll=4)
                    def _cp(i):
                        sl = pl.ds(i * LANES, LANES)
                        acc_v[sl] = tmp_v[sl]
                else:
                    @plsc.parallel_loop(0, seg // LANES, unroll=4)
                    def _add(i):
                        sl = pl.ds(i * LANES, LANES)
                        acc_v[sl] = acc_v[sl] + tmp_v[sl]
            pltpu.sync_copy(acc_v, dst_sh.at[pl.ds(off, seg)])
        plsc.subcore_barrier()
        pltpu.sync_copy(DGO, ho_v)
        pltpu.sync_copy(DGI, hi_v)

        base0 = pl.multiple_of(wid * ew, ew)
        pltpu.sync_copy(src_hbm.at[pl.ds(base0, ew)], sidx_v)
        pltpu.sync_copy(dst_hbm.at[pl.ds(base0, ew)], didx_v)

        def fire(k, ra, rb, sa, sb):
            o = pl.multiple_of(k * ch, ch)
            pltpu.async_copy(t_hbm.at[sidx_v.at[pl.ds(o, ch)]], ra, sa)
            pltpu.async_copy(t_hbm.at[didx_v.at[pl.ds(o, ch)]], rb, sb)

        def process(k, ra, rb, sa, sb, hp, sw):
            o = pl.multiple_of(k * ch, ch)
            pltpu.make_async_copy(t_hbm.at[sidx_v.at[pl.ds(o, ch)]], ra, sa).wait()
            pltpu.make_async_copy(t_hbm.at[didx_v.at[pl.ds(o, ch)]], rb, sb).wait()
            woff = pl.multiple_of((base0 + k * ch) // 4 * h2w, wlen)
            h1_dst = h1_hbm.at[pl.ds(woff, wlen)]

            @pl.when(k >= 2)
            def _():
                pltpu.make_async_copy(hp, h1_dst, sw).wait()

            @plsc.parallel_loop(0, ch // 4, unroll=2)
            def _row(p):
                r0 = 4 * p
                base = p * h2w
                for j in range(h // LANES):
                    sl = pl.ds(j * LANES, LANES)
                    sh = pl.ds(h + j * LANES, LANES)
                    v0 = ra[r0, sl] + rb[r0, sh]
                    v1 = ra[r0 + 1, sl] + rb[r0 + 1, sh]
                    v2 = ra[r0 + 2, sl] + rb[r0 + 2, sh]
                    v3 = ra[r0 + 3, sl] + rb[r0 + 3, sh]
                    hp[pl.ds(base + j * LANES, LANES)] = plsc.bitcast(
                        plsc.pack(v0, v1, format=plsc.PackFormat.INTERLEAVED),
                        jnp.int32)
                    hp[pl.ds(base + h + j * LANES, LANES)] = plsc.bitcast(
                        plsc.pack(v2, v3, format=plsc.PackFormat.INTERLEAVED),
                        jnp.int32)

            for j in range(ch // LANES):
                sl = pl.ds(o + j * LANES, LANES)
                do = jnp.maximum(plsc.load_gather(ho_v, [sidx_v[sl]]), fone)
                di = jnp.maximum(plsc.load_gather(hi_v, [didx_v[sl]]), fone)
                dp_v[sl] = do * di

            pltpu.async_copy(hp, h1_dst, sw)

        fire(0, ra0, rb0, sa0, sb0)

        @pl.loop(0, (nch - 1) // 2)
        def _g(g):
            k0 = 2 * g
            fire(k0 + 1, ra1, rb1, sa1, sb1)
            process(k0, ra0, rb0, sa0, sb0, hp0, sw0)
            fire(k0 + 2, ra0, rb0, sa0, sb0)
            process(k0 + 1, ra1, rb1, sa1, sb1, hp1, sw1)

        klast = nch - 1
        process(klast, ra0, rb0, sa0, sb0, hp0, sw0)
        w0d = pl.multiple_of((base0 + klast * ch) // 4 * h2w, wlen)
        pltpu.make_async_copy(hp0, h1_hbm.at[pl.ds(w0d, wlen)], sw0).wait()
        w1d = pl.multiple_of((base0 + (klast - 1) * ch) // 4 * h2w, wlen)
        pltpu.make_async_copy(hp1, h1_hbm.at[pl.ds(w1d, wlen)], sw1).wait()

        pltpu.sync_copy(dp_v, dp_hbm.at[pl.ds(base0, ew)])

    return main_k(src, dst, T)


def _tc_mlp(h1q, noise2, dp2, g1, be1, W2t, b2, g2, be2, W3r, b3,
            n_edges, h, h2, be_blk):
    nblk = (n_edges // 4) // be_blk

    def half_pipe(ht, g1v, be1v, w2v, b2v, g2v, be2v, w3v, b3v):
        m = jnp.mean(ht, axis=0, keepdims=True)
        v = jnp.mean((ht - m) ** 2, axis=0, keepdims=True)
        hn = (ht - m) * lax.rsqrt(v + EPS) * g1v + be1v
        hn = jnp.maximum(hn, 0.0)
        z = jnp.dot(w2v, hn, preferred_element_type=jnp.float32) + b2v
        m2 = jnp.mean(z, axis=0, keepdims=True)
        v2 = jnp.mean((z - m2) ** 2, axis=0, keepdims=True)
        zn = (z - m2) * lax.rsqrt(v2 + EPS) * g2v + be2v
        zn = jnp.maximum(zn, 0.0)
        return jnp.dot(w3v, zn, preferred_element_type=jnp.float32) + b3v

    def body(h1_ref, nz_ref, dp_ref, g1_ref, be1_ref, w2_ref, b2_ref, g2_ref,
             be2_ref, w3_ref, b3_ref, out_ref):
        g1v = g1_ref[...]
        be1v = be1_ref[...]
        w2v = w2_ref[...]
        b2v = b2_ref[...]
        g2v = g2_ref[...]
        be2v = be2_ref[...]
        w3v = w3_ref[...]
        b3v = b3_ref[...]
        htq = jnp.transpose(h1_ref[...])
        lo = lax.bitcast_convert_type(
            lax.shift_left(htq, 16), jnp.float32)
        hi = lax.bitcast_convert_type(
            jnp.bitwise_and(htq, jnp.int32(-65536)), jnp.float32)
        args = (g1v, be1v, w2v, b2v, g2v, be2v, w3v, b3v)
        ew0 = half_pipe(lo[:h, :], *args)
        ew1 = half_pipe(hi[:h, :], *args)
        ew2 = half_pipe(lo[h:, :], *args)
        ew3 = half_pipe(hi[h:, :], *args)
        ew = jnp.concatenate([ew0, ew1, ew2, ew3], axis=0)
        nz = nz_ref[0]
        gate = jnp.log(nz) - jnp.log(1.0 - nz)
        mask = 1.0 / (1.0 + jnp.exp(-(gate + ew)))
        out_ref[0] = mask * lax.rsqrt(dp_ref[0])

    edge_spec = pl.BlockSpec((1, 4, be_blk), lambda i: (i, 0, 0))
    return pl.pallas_call(
        body,
        grid=(nblk,),
        in_specs=[
            pl.BlockSpec((be_blk, 2 * h), lambda i: (i, 0)),
            edge_spec,
            edge_spec,
            pl.BlockSpec((h, 1), lambda i: (0, 0)),
            pl.BlockSpec((h, 1), lambda i: (0, 0)),
            pl.BlockSpec((h2, h), lambda i: (0, 0)),
            pl.BlockSpec((h2, 1), lambda i: (0, 0)),
            pl.BlockSpec((h2, 1), lambda i: (0, 0)),
            pl.BlockSpec((h2, 1), lambda i: (0, 0)),
            pl.BlockSpec((1, h2), lambda i: (0, 0)),
            pl.BlockSpec((1, 1), lambda i: (0, 0)),
        ],
        out_specs=edge_spec,
        out_shape=jax.ShapeDtypeStruct((nblk, 4, be_blk), jnp.float32),
    )(h1q, noise2, dp2, g1, be1, W2t, b2, g2, be2, W3r, b3)


def kernel(node_embeddings, edge_index, W1, b1, g1, be1, W2, b2, g2, be2, W3, b3):
    n_nodes, d = node_embeddings.shape
    n_edges = edge_index.shape[1]
    h = W1.shape[1]
    h2 = W2.shape[1]

    info = plsc.get_sparse_core_info()
    ch = 80

    src = edge_index[0]
    dst = edge_index[1]

    Wcat = jnp.concatenate([W1[:d], W1[d:]], axis=1)
    bcat = jnp.concatenate([b1, jnp.zeros_like(b1)]).reshape(1, 2 * h)

    T = _tc_table(node_embeddings, Wcat, bcat, n_nodes, h)
    h1f, dp = _sc_main(src, dst, T, n_nodes, n_edges, h,
                       info.num_cores, info.num_subcores, ch)
    h1q = h1f.reshape(n_edges // 4, 2 * h)

    noise = jax.random.uniform(jax.random.key(42), (n_edges,),
                               dtype=jnp.float32, minval=1e-6, maxval=1.0 - 1e-6)

    be_blk = 8000
    nblk = (n_edges // 4) // be_blk
    noise2 = noise.reshape(nblk, be_blk, 4).transpose(0, 2, 1)
    dp2 = dp.reshape(nblk, be_blk, 4).transpose(0, 2, 1)
    out2 = _tc_mlp(
        h1q, noise2, dp2,
        g1.reshape(h, 1), be1.reshape(h, 1),
        W2.T, b2.reshape(h2, 1), g2.reshape(h2, 1), be2.reshape(h2, 1),
        W3.reshape(1, h2), b3.reshape(1, 1),
        n_edges, h, h2, be_blk,
    )
    return out2.transpose(0, 2, 1).reshape(n_edges)
